# baseline (speedup 1.0000x reference)
import jax
import jax.numpy as jnp
from jax import lax
from jax.experimental import pallas as pl
from jax.experimental.pallas import tpu as pltpu
from jax.experimental.pallas import tpu_sc as plsc

N = 10000
NP = 10240
E = 160000
EH = E // 2
H = 8
F = 2 * H * H
NC = 2
NS = 16
NW = NC * NS
NPT = NP // NS
NB = 2

EPWS = EH // NW
CHS = 50
NITS = EPWS // CHS

CHG = 250
CHG2 = CHG // 2
NITG = EPWS // CHG

_f32 = jnp.float32
_MESH = dict(core_axis_name="c", subcore_axis_name="s")


def _sc_scatter(payload3, dstidx3):
    def body(p_hbm, idx_hbm, z_hbm, out_hbm, idx_sl, b0, b1, l0, l1, s0, s1,
             acc_sh):
        bufs = (b0, b1)
        lsems = (l0, l1)
        ssems = (s0, s1)
        c = lax.axis_index("c")
        s = lax.axis_index("s")
        wid = c * NS + s
        chunk0 = wid * NITS
        nsl = pl.ds(s * NPT, NPT)
        pltpu.sync_copy(z_hbm.at[nsl], acc_sh.at[nsl])
        pltpu.sync_copy(idx_hbm.at[wid], idx_sl)
        plsc.subcore_barrier()

        def load(i, b):
            pltpu.async_copy(p_hbm.at[chunk0 + i], bufs[b], lsems[b])

        for b in range(NB - 1):
            load(b, b)

        def outer(g, carry):
            i0 = g * NB
            for b in range(NB):
                i = i0 + b
                pltpu.make_async_copy(p_hbm.at[chunk0 + i], bufs[b],
                                      lsems[b]).wait()
                pltpu.async_copy(bufs[b], acc_sh.at[idx_sl.at[i]], ssems[b],
                                 add=True)
                nxt = i + NB - 1
                bn = (b + NB - 1) % NB

                @pl.when(nxt < NITS)
                def _():
                    @pl.when(i >= 1)
                    def _():
                        pltpu.make_async_copy(
                            bufs[bn], acc_sh.at[idx_sl.at[i - 1]],
                            ssems[bn]).wait()
                    load(nxt, bn)
            return carry

        lax.fori_loop(0, NITS // NB, outer, 0)
        for b in range(NB):
            i = NITS - NB + b
            pltpu.make_async_copy(bufs[b], acc_sh.at[idx_sl.at[i]],
                                  ssems[b]).wait()
        plsc.subcore_barrier()
        pltpu.sync_copy(acc_sh.at[nsl], out_hbm.at[pl.ds(c * NP + s * NPT, NPT)])

    zeros = jnp.zeros((NP, F), _f32)
    return pl.kernel(
        body,
        out_type=jax.ShapeDtypeStruct((2 * NP, F), _f32),
        mesh=plsc.VectorSubcoreMesh(**_MESH),
        scratch_types=(
            [pltpu.VMEM((NITS, CHS), jnp.int32)]
            + [pltpu.VMEM((CHS, F), _f32) for _ in range(NB)]
            + [pltpu.SemaphoreType.DMA] * (2 * NB)
            + [pltpu.VMEM_SHARED((NP, F), _f32)]
        ),
    )(payload3, dstidx3, zeros)


def _sc_gather(table, srcidx4):
    def body(t_hbm, idx_hbm, out_hbm, idx_sl, b0, b1, g0, g1, t0, t1):
        bufs = (b0, b1)
        gsems = (g0, g1)
        stsems = (t0, t1)
        c = lax.axis_index("c")
        s = lax.axis_index("s")
        wid = c * NS + s
        chunk0 = wid * NITG
        pltpu.sync_copy(idx_hbm.at[wid], idx_sl)

        def gather(i, bi):
            pltpu.async_copy(t_hbm.at[idx_sl.at[i, 0]],
                             bufs[bi].at[pl.ds(0, CHG2)], gsems[bi])
            pltpu.async_copy(t_hbm.at[idx_sl.at[i, 1]],
                             bufs[bi].at[pl.ds(CHG2, CHG2)], gsems[bi])

        def wait_gather(i, bi):
            pltpu.make_async_copy(t_hbm.at[idx_sl.at[i, 0]],
                                  bufs[bi].at[pl.ds(0, CHG2)], gsems[bi]).wait()
            pltpu.make_async_copy(t_hbm.at[idx_sl.at[i, 1]],
                                  bufs[bi].at[pl.ds(CHG2, CHG2)],
                                  gsems[bi]).wait()

        def store(i, bi):
            pltpu.async_copy(bufs[bi], out_hbm.at[chunk0 + i], stsems[bi])

        def wait_store(i, bi):
            pltpu.make_async_copy(bufs[bi], out_hbm.at[chunk0 + i],
                                  stsems[bi]).wait()

        gather(0, 0)

        def outer(g, carry):
            i = 2 * g
            wait_gather(i, 0)

            @pl.when(g >= 1)
            def _():
                wait_store(i - 1, 1)

            gather(i + 1, 1)
            store(i, 0)
            wait_gather(i + 1, 1)

            @pl.when(i + 2 < NITG)
            def _():
                wait_store(i, 0)
                gather(i + 2, 0)

            store(i + 1, 1)
            return carry

        lax.fori_loop(0, NITG // 2, outer, 0)
        wait_store(NITG - 2, 0)
        wait_store(NITG - 1, 1)

    return pl.kernel(
        body,
        out_type=jax.ShapeDtypeStruct((NW * NITG, CHG, F), _f32),
        mesh=plsc.VectorSubcoreMesh(**_MESH),
        scratch_types=(
            [pltpu.VMEM((NITG, 2, CHG2), jnp.int32)]
            + [pltpu.VMEM((CHG, F), _f32) for _ in range(2)]
            + [pltpu.SemaphoreType.DMA] * 4
        ),
    )(table, srcidx4)


BE = 2000
GEH = EH // BE
CPBS = BE // CHS
CPBG = BE // CHG


def _full(shape):
    nd = len(shape)
    return pl.BlockSpec(shape, lambda i: (0,) * nd)


def _blk(shape):
    return pl.BlockSpec(shape, lambda i: (i,) + (0,) * (len(shape) - 1))


def _node_proj(node_feats, wab):
    def body(nf_ref, w_ref, out_ref):
        ab = jnp.dot(nf_ref[...], w_ref[...], preferred_element_type=_f32)
        out_ref[...] = jnp.concatenate(
            [ab, jnp.zeros((ab.shape[0], F - 2 * H), _f32)], axis=1)

    return pl.pallas_call(
        body,
        grid=(10,),
        in_specs=[_blk((N // 10, 128)), _full((128, 2 * H))],
        out_specs=_blk((N // 10, F)),
        out_shape=jax.ShapeDtypeStruct((NP, F), _f32),
    )(node_feats, wab)


def _init_ef(g_s, g_d, edge_feats, wc):
    def body(s_ref, d_ref, ef_ref, w_ref, out_ref):
        gs = s_ref[...].reshape(BE, F)
        gd = d_ref[...].reshape(BE, F)
        x = (gs[:, :H] + gd[:, H:2 * H]
             + jnp.dot(ef_ref[...], w_ref[...], preferred_element_type=_f32))
        x = jnp.maximum(x, 0.0)
        out_ref[...] = jnp.concatenate([x, jnp.zeros_like(x)], axis=1)

    return pl.pallas_call(
        body,
        grid=(GEH,),
        in_specs=[_blk((CPBG, CHG, F)), _blk((CPBG, CHG, F)), _blk((BE, 16)),
                  _full((16, H))],
        out_specs=_blk((BE, 2 * H)),
        out_shape=jax.ShapeDtypeStruct((EH, 2 * H), _f32),
    )(g_s, g_d, edge_feats, wc)


def _edge_mats(ef16, w1cat, b1cat, w2blk, b2cat, rm2):
    t = jnp.dot(ef16, w1cat, preferred_element_type=_f32) + b1cat
    u = jnp.maximum(t, 0.0)
    wma = jnp.dot(u, w2blk, preferred_element_type=_f32) + b2cat
    ef_r2 = jnp.dot(ef16, rm2, preferred_element_type=_f32)
    return wma, wma * ef_r2


def _pass1(ef16, w1cat, b1cat, w2blk, b2cat, rm2):
    def body(ef_ref, w1_ref, b1_ref, w2_ref, b2_ref, rm2_ref, out_ref):
        _, e_all = _edge_mats(ef_ref[...], w1_ref[...], b1_ref[...],
                              w2_ref[...], b2_ref[...], rm2_ref[...])
        exp_e2 = jnp.exp(e_all[:, H * H:])
        h1 = exp_e2 * e_all[:, :H * H]
        out_ref[...] = jnp.concatenate([exp_e2, h1],
                                       axis=1).reshape(CPBS, CHS, F)

    return pl.pallas_call(
        body,
        grid=(GEH,),
        in_specs=[_blk((BE, 2 * H)), _full((2 * H, 2 * H)), _full((1, 2 * H)),
                  _full((2 * H, F)), _full((1, F)), _full((2 * H, F))],
        out_specs=_blk((CPBS, CHS, F)),
        out_shape=jax.ShapeDtypeStruct((NW * NITS, CHS, F), _f32),
    )(ef16, w1cat, b1cat, w2blk, b2cat, rm2)


def _combine_partials(pa, pb):
    def body(a_ref, b_ref, c_ref, d_ref, out_ref):
        out_ref[...] = (a_ref[...] + b_ref[...]) + (c_ref[...] + d_ref[...])

    return pl.pallas_call(
        body,
        grid=(10,),
        in_specs=[
            pl.BlockSpec((NP // 10, F), lambda i: (i, 0)),
            pl.BlockSpec((NP // 10, F), lambda i: (i + 10, 0)),
            pl.BlockSpec((NP // 10, F), lambda i: (i, 0)),
            pl.BlockSpec((NP // 10, F), lambda i: (i + 10, 0)),
        ],
        out_specs=pl.BlockSpec((NP // 10, F), lambda i: (i, 0)),
        out_shape=jax.ShapeDtypeStruct((NP, F), _f32),
    )(pa, pa, pb, pb)


def _pass2_gru(g, ef16, ief16, cw, out_3d=False):
    def body(g_ref, ef_ref, ief_ref, w1_ref, b1_ref, w2_ref, b2_ref, rm2_ref,
             rt_ref, wir_ref, wiz_ref, win_ref, whr_ref, whz_ref, whn_ref,
             gb_ref, out_ref):
        ef16v = ef_ref[...]
        wma, e_all = _edge_mats(ef16v, w1_ref[...], b1_ref[...], w2_ref[...],
                                b2_ref[...], rm2_ref[...])
        ie_all = wma * jnp.dot(ief_ref[...], rm2_ref[...],
                               preferred_element_type=_f32)
        exp_e2 = jnp.exp(e_all[:, H * H:])
        h1 = exp_e2 * e_all[:, :H * H]
        exp_ie2 = jnp.exp(ie_all[:, H * H:])
        ih1 = exp_ie2 * ie_all[:, :H * H]
        gathered = g_ref[...].reshape(BE, F)
        sg = gathered[:, :H * H]
        mg = gathered[:, H * H:]
        h2 = (mg - h1 + ih1) / (sg - exp_e2 + exp_ie2)
        conv = jnp.maximum(jnp.dot(h2, rt_ref[...],
                                   preferred_element_type=_f32), 0.0)
        gb = gb_ref[...]
        r = jax.nn.sigmoid(
            jnp.dot(conv, wir_ref[...], preferred_element_type=_f32)
            + jnp.dot(ef16v, whr_ref[...], preferred_element_type=_f32)
            + gb[:, :H])
        z = jax.nn.sigmoid(
            jnp.dot(conv, wiz_ref[...], preferred_element_type=_f32)
            + jnp.dot(ef16v, whz_ref[...], preferred_element_type=_f32)
            + gb[:, H:2 * H])
        n = jnp.tanh(
            jnp.dot(conv, win_ref[...], preferred_element_type=_f32)
            + gb[:, 2 * H:3 * H]
            + r * (jnp.dot(ef16v, whn_ref[...], preferred_element_type=_f32)
                   + gb[:, 3 * H:]))
        newef = (1.0 - z) * n + z * ef16v[:, :H]
        if out_3d:
            out_ref[...] = jnp.concatenate(
                [newef, jnp.zeros((BE, F - H), _f32)],
                axis=1).reshape(CPBS, CHS, F)
        else:
            out_ref[...] = jnp.concatenate([newef, jnp.zeros_like(newef)],
                                           axis=1)

    if out_3d:
        out_spec = _blk((CPBS, CHS, F))
        out_shape = jax.ShapeDtypeStruct((NW * NITS, CHS, F), _f32)
    else:
        out_spec = _blk((BE, 2 * H))
        out_shape = jax.ShapeDtypeStruct((EH, 2 * H), _f32)
    (w1cat, b1cat, w2blk, b2cat, rm2, rt, wir, wiz, win, whr, whz, whn,
     gbias) = cw
    return pl.pallas_call(
        body,
        grid=(GEH,),
        in_specs=[_blk((CPBG, CHG, F)), _blk((BE, 2 * H)), _blk((BE, 2 * H)),
                  _full((2 * H, 2 * H)), _full((1, 2 * H)), _full((2 * H, F)),
                  _full((1, F)), _full((2 * H, F)), _full((H * H, H)),
                  _full((H, H)), _full((H, H)), _full((H, H)),
                  _full((2 * H, H)), _full((2 * H, H)), _full((2 * H, H)),
                  _full((1, 4 * H))],
        out_specs=out_spec,
        out_shape=out_shape,
    )(g, ef16, ief16, w1cat, b1cat, w2blk, b2cat, rm2, rt, wir, wiz, win,
      whr, whz, whn, gbias)


def _readout(pa, pb):
    def body(a_ref, b_ref, c_ref, d_ref, out_ref):
        out_ref[...] = ((a_ref[...] + b_ref[...])
                        + (c_ref[...] + d_ref[...]))[:, :H]

    return pl.pallas_call(
        body,
        grid=(10,),
        in_specs=[
            pl.BlockSpec((NP // 10, F), lambda i: (i, 0)),
            pl.BlockSpec((NP // 10, F), lambda i: (i + 10, 0)),
            pl.BlockSpec((NP // 10, F), lambda i: (i, 0)),
            pl.BlockSpec((NP // 10, F), lambda i: (i + 10, 0)),
        ],
        out_specs=pl.BlockSpec((NP // 10, H), lambda i: (i, 0)),
        out_shape=jax.ShapeDtypeStruct((NP, H), _f32),
    )(pa, pa, pb, pb)


def kernel(node_feats, edge_feats, edge_index, W_i, msg_W1, msg_b1, msg_W2,
           msg_b2, attn_W1, attn_b1, attn_W2, attn_b2, gru_Wih, gru_bih,
           gru_Whh, gru_bhh):
    src = edge_index[0]
    dst = edge_index[1]
    srcg = [src[:EH].reshape(NW, NITG, 2, CHG2),
            src[EH:].reshape(NW, NITG, 2, CHG2)]
    dstg = [dst[:EH].reshape(NW, NITG, 2, CHG2),
            dst[EH:].reshape(NW, NITG, 2, CHG2)]
    dsts = [dst[:EH].reshape(NW, NITS, CHS),
            dst[EH:].reshape(NW, NITS, CHS)]
    efh = [edge_feats[:EH], edge_feats[EH:]]

    wab = jnp.concatenate([W_i[:128], W_i[128:256]], axis=1)
    wc = W_i[256:]
    rm = jnp.repeat(jnp.eye(H, dtype=_f32), H, axis=1)
    zrow = jnp.zeros((H, H), _f32)
    w1cat = jnp.concatenate([
        jnp.concatenate([msg_W1, attn_W1], axis=1),
        jnp.zeros((H, 2 * H), _f32)], axis=0)
    b1cat = jnp.concatenate([msg_b1, attn_b1]).reshape(1, 2 * H)
    w2blk = jnp.concatenate([
        jnp.concatenate([msg_W2, jnp.zeros((H, H * H), _f32)], axis=1),
        jnp.concatenate([jnp.zeros((H, H * H), _f32), attn_W2], axis=1),
    ], axis=0)
    b2cat = jnp.concatenate([msg_b2, attn_b2]).reshape(1, F)
    rm2 = jnp.concatenate([
        jnp.concatenate([rm, rm], axis=1),
        jnp.zeros((H, F), _f32)], axis=0)
    rt = rm.T
    wir, wiz, win = (gru_Wih[:, :H], gru_Wih[:, H:2 * H], gru_Wih[:, 2 * H:])
    whr = jnp.concatenate([gru_Whh[:, :H], zrow], axis=0)
    whz = jnp.concatenate([gru_Whh[:, H:2 * H], zrow], axis=0)
    whn = jnp.concatenate([gru_Whh[:, 2 * H:], zrow], axis=0)
    gbias = jnp.concatenate([
        gru_bih[:H] + gru_bhh[:H],
        gru_bih[H:2 * H] + gru_bhh[H:2 * H],
        gru_bih[2 * H:],
        gru_bhh[2 * H:]]).reshape(1, 4 * H)
    cw = (w1cat, b1cat, w2blk, b2cat, rm2, rt, wir, wiz, win, whr, whz, whn,
          gbias)

    t128 = _node_proj(node_feats, wab)
    ef = [None, None]
    for hh in range(2):
        g_s = _sc_gather(t128, srcg[hh])
        g_d = _sc_gather(t128, dstg[hh])
        ef[hh] = _init_ef(g_s, g_d, efh[hh], wc)
    ief = list(ef)

    newef = ef
    for step in range(3):
        pay = [_pass1(ef[hh], w1cat, b1cat, w2blk, b2cat, rm2)
               for hh in range(2)]
        parts = [_sc_scatter(pay[hh], dsts[hh]) for hh in range(2)]
        sm = _combine_partials(parts[0], parts[1])
        gs = [_sc_gather(sm, srcg[hh]) for hh in range(2)]
        newef = [_pass2_gru(gs[hh], ef[hh], ief[hh], cw,
                            out_3d=(step == 2)) for hh in range(2)]
        if step < 2:
            ef = newef

    parts = [_sc_scatter(newef[hh], dsts[hh]) for hh in range(2)]
    return _readout(parts[0], parts[1])[:N]


# speedup vs baseline: 1.3009x; 1.3009x over previous
import jax
import jax.numpy as jnp
from jax import lax
from jax.experimental import pallas as pl
from jax.experimental.pallas import tpu as pltpu
from jax.experimental.pallas import tpu_sc as plsc

N = 10000
NP = 10240
E = 160000
H = 8
F = 2 * H * H
NC = 2
NS = 16
NW = NC * NS
EPW = E // NW
CHX = 100
NITX = EPW // CHX
NB = 2
CHB = 200
CH2 = 100
NIT2 = EPW // CHB
NPT = NP // NS

_f32 = jnp.float32
_MESH = dict(core_axis_name="c", subcore_axis_name="s")


def _sc_scatter(payload3, dstidx3, f):
    def body(p_hbm, idx_hbm, z_hbm, out_hbm, idx_sl, b0, b1,
             l0, l1, s0, s1, acc_sh):
        bufs = (b0, b1)
        lsems = (l0, l1)
        ssems = (s0, s1)
        c = lax.axis_index("c")
        s = lax.axis_index("s")
        wid = c * NS + s
        chunk0 = wid * NITX
        nsl = pl.ds(s * NPT, NPT)
        pltpu.sync_copy(z_hbm.at[nsl], acc_sh.at[nsl])
        pltpu.sync_copy(idx_hbm.at[wid], idx_sl)
        plsc.subcore_barrier()

        def load(i, b):
            pltpu.async_copy(p_hbm.at[chunk0 + i], bufs[b], lsems[b])

        for b in range(NB - 1):
            load(b, b)

        def outer(g, carry):
            i0 = g * NB
            for b in range(NB):
                i = i0 + b
                pltpu.make_async_copy(p_hbm.at[chunk0 + i], bufs[b],
                                      lsems[b]).wait()
                pltpu.async_copy(bufs[b], acc_sh.at[idx_sl.at[i]], ssems[b],
                                 add=True)
                nxt = i + NB - 1
                bn = (b + NB - 1) % NB

                @pl.when(nxt < NITX)
                def _():
                    @pl.when(i >= 1)
                    def _():
                        pltpu.make_async_copy(
                            bufs[bn], acc_sh.at[idx_sl.at[i - 1]],
                            ssems[bn]).wait()
                    load(nxt, bn)
            return carry

        lax.fori_loop(0, NITX // NB, outer, 0)
        for b in range(NB):
            i = NITX - NB + b
            pltpu.make_async_copy(bufs[b], acc_sh.at[idx_sl.at[i]],
                                  ssems[b]).wait()
        plsc.subcore_barrier()
        pltpu.sync_copy(acc_sh.at[nsl], out_hbm.at[pl.ds(c * NP + s * NPT, NPT)])

    zeros = jnp.zeros((NP, f), _f32)
    return pl.kernel(
        body,
        out_type=jax.ShapeDtypeStruct((2 * NP, f), _f32),
        mesh=plsc.VectorSubcoreMesh(**_MESH),
        scratch_types=(
            [pltpu.VMEM((NITX, CHX), jnp.int32)]
            + [pltpu.VMEM((CHX, f), _f32) for _ in range(NB)]
            + [pltpu.SemaphoreType.DMA] * (2 * NB)
            + [pltpu.VMEM_SHARED((NP, f), _f32)]
        ),
    )(payload3, dstidx3, zeros)


def _sc_gather(table, srcidx4, f):
    def body(t_hbm, idx_hbm, out_hbm, idx_sl, b0, b1, g0, g1, t0, t1):
        bufs = (b0, b1)
        gsems = (g0, g1)
        stsems = (t0, t1)
        c = lax.axis_index("c")
        s = lax.axis_index("s")
        wid = c * NS + s
        ebase = wid * EPW
        pltpu.sync_copy(idx_hbm.at[wid], idx_sl)

        def gather(i, bi):
            pltpu.async_copy(t_hbm.at[idx_sl.at[i, 0]],
                             bufs[bi].at[pl.ds(0, CH2)], gsems[bi])
            pltpu.async_copy(t_hbm.at[idx_sl.at[i, 1]],
                             bufs[bi].at[pl.ds(CH2, CH2)], gsems[bi])

        def wait_gather(i, bi):
            pltpu.make_async_copy(t_hbm.at[idx_sl.at[i, 0]],
                                  bufs[bi].at[pl.ds(0, CH2)], gsems[bi]).wait()
            pltpu.make_async_copy(t_hbm.at[idx_sl.at[i, 1]],
                                  bufs[bi].at[pl.ds(CH2, CH2)], gsems[bi]).wait()

        def store(i, bi):
            pltpu.async_copy(bufs[bi], out_hbm.at[pl.ds(ebase + i * CHB, CHB)],
                             stsems[bi])

        def wait_store(i, bi):
            pltpu.make_async_copy(bufs[bi],
                                  out_hbm.at[pl.ds(ebase + i * CHB, CHB)],
                                  stsems[bi]).wait()

        gather(0, 0)

        def outer(g, carry):
            i = 2 * g
            wait_gather(i, 0)

            @pl.when(g >= 1)
            def _():
                wait_store(i - 1, 1)

            gather(i + 1, 1)
            store(i, 0)
            wait_gather(i + 1, 1)

            @pl.when(i + 2 < NIT2)
            def _():
                wait_store(i, 0)
                gather(i + 2, 0)

            store(i + 1, 1)
            return carry

        lax.fori_loop(0, NIT2 // 2, outer, 0)
        wait_gather(NIT2 - 1, 0)
        store(NIT2 - 1, 0)
        wait_store(NIT2 - 2, 1)
        wait_store(NIT2 - 1, 0)

    return pl.kernel(
        body,
        out_type=jax.ShapeDtypeStruct((E, f), _f32),
        mesh=plsc.VectorSubcoreMesh(**_MESH),
        scratch_types=(
            [pltpu.VMEM((NIT2, 2, CH2), jnp.int32)]
            + [pltpu.VMEM((CHB, f), _f32) for _ in range(2)]
            + [pltpu.SemaphoreType.DMA] * 4
        ),
    )(table, srcidx4)


BE = 2000
GE = E // BE
CPB = BE // CHX


def _full(shape):
    nd = len(shape)
    return pl.BlockSpec(shape, lambda i: (0,) * nd)


def _blk(shape):
    return pl.BlockSpec(shape, lambda i: (i,) + (0,) * (len(shape) - 1))


def _node_proj(node_feats, wab):
    def body(nf_ref, w_ref, out_ref):
        ab = jnp.dot(nf_ref[...], w_ref[...], preferred_element_type=_f32)
        out_ref[...] = jnp.concatenate(
            [ab, jnp.zeros((ab.shape[0], F - 2 * H), _f32)], axis=1)

    return pl.pallas_call(
        body,
        grid=(10,),
        in_specs=[_blk((N // 10, 128)), _full((128, 2 * H))],
        out_specs=_blk((N // 10, F)),
        out_shape=jax.ShapeDtypeStruct((NP, F), _f32),
    )(node_feats, wab)


def _init_ef(g_s, g_d, edge_feats, wc):
    def body(s_ref, d_ref, ef_ref, w_ref, out_ref):
        x = (s_ref[...][:, :H] + d_ref[...][:, H:2 * H]
             + jnp.dot(ef_ref[...], w_ref[...], preferred_element_type=_f32))
        x = jnp.maximum(x, 0.0)
        out_ref[...] = jnp.concatenate([x, jnp.zeros_like(x)], axis=1)

    return pl.pallas_call(
        body,
        grid=(GE,),
        in_specs=[_blk((BE, F)), _blk((BE, F)), _blk((BE, 16)),
                  _full((16, H))],
        out_specs=_blk((BE, 2 * H)),
        out_shape=jax.ShapeDtypeStruct((E, 2 * H), _f32),
    )(g_s, g_d, edge_feats, wc)


def _edge_mats(ef16, w1cat, b1cat, w2blk, b2cat, rm2):
    t = jnp.dot(ef16, w1cat, preferred_element_type=_f32) + b1cat
    u = jnp.maximum(t, 0.0)
    wma = jnp.dot(u, w2blk, preferred_element_type=_f32) + b2cat
    ef_r2 = jnp.dot(ef16, rm2, preferred_element_type=_f32)
    return wma, wma * ef_r2


def _pass1(ef16, w1cat, b1cat, w2blk, b2cat, rm2):
    def body(ef_ref, w1_ref, b1_ref, w2_ref, b2_ref, rm2_ref, out_ref):
        _, e_all = _edge_mats(ef_ref[...], w1_ref[...], b1_ref[...],
                              w2_ref[...], b2_ref[...], rm2_ref[...])
        exp_e2 = jnp.exp(e_all[:, H * H:])
        h1 = exp_e2 * e_all[:, :H * H]
        out_ref[...] = jnp.concatenate([exp_e2, h1],
                                       axis=1).reshape(CPB, CHX, F)

    return pl.pallas_call(
        body,
        grid=(GE,),
        in_specs=[_blk((BE, 2 * H)), _full((2 * H, 2 * H)), _full((1, 2 * H)),
                  _full((2 * H, F)), _full((1, F)), _full((2 * H, F))],
        out_specs=_blk((CPB, CHX, F)),
        out_shape=jax.ShapeDtypeStruct((NW * NITX, CHX, F), _f32),
    )(ef16, w1cat, b1cat, w2blk, b2cat, rm2)


def _combine_partials(partials, f):
    def body(a_ref, b_ref, out_ref):
        out_ref[...] = a_ref[...] + b_ref[...]

    return pl.pallas_call(
        body,
        grid=(10,),
        in_specs=[
            pl.BlockSpec((NP // 10, f), lambda i: (i, 0)),
            pl.BlockSpec((NP // 10, f), lambda i: (i + 10, 0)),
        ],
        out_specs=pl.BlockSpec((NP // 10, f), lambda i: (i, 0)),
        out_shape=jax.ShapeDtypeStruct((NP, f), _f32),
    )(partials, partials)


def _pass2_gru(g, ef16, ief16, cw, out_3d=False):
    def body(g_ref, ef_ref, ief_ref, w1_ref, b1_ref, w2_ref, b2_ref, rm2_ref,
             rt_ref, wir_ref, wiz_ref, win_ref, whr_ref, whz_ref, whn_ref,
             gb_ref, out_ref):
        ef16v = ef_ref[...]
        wma, e_all = _edge_mats(ef16v, w1_ref[...], b1_ref[...], w2_ref[...],
                                b2_ref[...], rm2_ref[...])
        ie_all = wma * jnp.dot(ief_ref[...], rm2_ref[...],
                               preferred_element_type=_f32)
        exp_e2 = jnp.exp(e_all[:, H * H:])
        h1 = exp_e2 * e_all[:, :H * H]
        exp_ie2 = jnp.exp(ie_all[:, H * H:])
        ih1 = exp_ie2 * ie_all[:, :H * H]
        gathered = g_ref[...]
        sg = gathered[:, :H * H]
        mg = gathered[:, H * H:]
        h2 = (mg - h1 + ih1) / (sg - exp_e2 + exp_ie2)
        conv = jnp.maximum(jnp.dot(h2, rt_ref[...],
                                   preferred_element_type=_f32), 0.0)
        gb = gb_ref[...]
        r = jax.nn.sigmoid(
            jnp.dot(conv, wir_ref[...], preferred_element_type=_f32)
            + jnp.dot(ef16v, whr_ref[...], preferred_element_type=_f32)
            + gb[:, :H])
        z = jax.nn.sigmoid(
            jnp.dot(conv, wiz_ref[...], preferred_element_type=_f32)
            + jnp.dot(ef16v, whz_ref[...], preferred_element_type=_f32)
            + gb[:, H:2 * H])
        n = jnp.tanh(
            jnp.dot(conv, win_ref[...], preferred_element_type=_f32)
            + gb[:, 2 * H:3 * H]
            + r * (jnp.dot(ef16v, whn_ref[...], preferred_element_type=_f32)
                   + gb[:, 3 * H:]))
        newef = (1.0 - z) * n + z * ef16v[:, :H]
        if out_3d:
            out_ref[...] = jnp.concatenate(
                [newef, jnp.zeros((BE, F - H), _f32)],
                axis=1).reshape(CPB, CHX, F)
        else:
            out_ref[...] = jnp.concatenate([newef, jnp.zeros_like(newef)],
                                           axis=1)

    if out_3d:
        out_spec = _blk((CPB, CHX, F))
        out_shape = jax.ShapeDtypeStruct((NW * NITX, CHX, F), _f32)
    else:
        out_spec = _blk((BE, 2 * H))
        out_shape = jax.ShapeDtypeStruct((E, 2 * H), _f32)
    (w1cat, b1cat, w2blk, b2cat, rm2, rt, wir, wiz, win, whr, whz, whn,
     gbias) = cw
    return pl.pallas_call(
        body,
        grid=(GE,),
        in_specs=[_blk((BE, F)), _blk((BE, 2 * H)), _blk((BE, 2 * H)),
                  _full((2 * H, 2 * H)), _full((1, 2 * H)), _full((2 * H, F)),
                  _full((1, F)), _full((2 * H, F)), _full((H * H, H)),
                  _full((H, H)), _full((H, H)), _full((H, H)),
                  _full((2 * H, H)), _full((2 * H, H)), _full((2 * H, H)),
                  _full((1, 4 * H))],
        out_specs=out_spec,
        out_shape=out_shape,
    )(g, ef16, ief16, w1cat, b1cat, w2blk, b2cat, rm2, rt, wir, wiz, win,
      whr, whz, whn, gbias)


def _readout(partials):
    def body(a_ref, b_ref, out_ref):
        out_ref[...] = (a_ref[...] + b_ref[...])[:, :H]

    return pl.pallas_call(
        body,
        grid=(10,),
        in_specs=[
            pl.BlockSpec((NP // 10, F), lambda i: (i, 0)),
            pl.BlockSpec((NP // 10, F), lambda i: (i + 10, 0)),
        ],
        out_specs=pl.BlockSpec((NP // 10, H), lambda i: (i, 0)),
        out_shape=jax.ShapeDtypeStruct((NP, H), _f32),
    )(partials, partials)


def kernel(node_feats, edge_feats, edge_index, W_i, msg_W1, msg_b1, msg_W2,
           msg_b2, attn_W1, attn_b1, attn_W2, attn_b2, gru_Wih, gru_bih,
           gru_Whh, gru_bhh):
    src4 = edge_index[0].reshape(NW, NIT2, 2, CH2)
    dst4 = edge_index[1].reshape(NW, NIT2, 2, CH2)
    dst3 = edge_index[1].reshape(NW, NITX, CHX)

    wab = jnp.concatenate([W_i[:128], W_i[128:256]], axis=1)
    wc = W_i[256:]
    rm = jnp.repeat(jnp.eye(H, dtype=_f32), H, axis=1)
    zrow = jnp.zeros((H, H), _f32)
    w1cat = jnp.concatenate([
        jnp.concatenate([msg_W1, attn_W1], axis=1),
        jnp.zeros((H, 2 * H), _f32)], axis=0)
    b1cat = jnp.concatenate([msg_b1, attn_b1]).reshape(1, 2 * H)
    w2blk = jnp.concatenate([
        jnp.concatenate([msg_W2, jnp.zeros((H, H * H), _f32)], axis=1),
        jnp.concatenate([jnp.zeros((H, H * H), _f32), attn_W2], axis=1),
    ], axis=0)
    b2cat = jnp.concatenate([msg_b2, attn_b2]).reshape(1, F)
    rm2 = jnp.concatenate([
        jnp.concatenate([rm, rm], axis=1),
        jnp.zeros((H, F), _f32)], axis=0)
    rt = rm.T
    wir, wiz, win = (gru_Wih[:, :H], gru_Wih[:, H:2 * H], gru_Wih[:, 2 * H:])
    whr = jnp.concatenate([gru_Whh[:, :H], zrow], axis=0)
    whz = jnp.concatenate([gru_Whh[:, H:2 * H], zrow], axis=0)
    whn = jnp.concatenate([gru_Whh[:, 2 * H:], zrow], axis=0)
    gbias = jnp.concatenate([
        gru_bih[:H] + gru_bhh[:H],
        gru_bih[H:2 * H] + gru_bhh[H:2 * H],
        gru_bih[2 * H:],
        gru_bhh[2 * H:]]).reshape(1, 4 * H)
    cw = (w1cat, b1cat, w2blk, b2cat, rm2, rt, wir, wiz, win, whr, whz, whn,
          gbias)

    t128 = _node_proj(node_feats, wab)
    g_s = _sc_gather(t128, src4, F)
    g_d = _sc_gather(t128, dst4, F)
    ef16 = _init_ef(g_s, g_d, edge_feats, wc)
    ief16 = ef16

    for step in range(3):
        payload = _pass1(ef16, w1cat, b1cat, w2blk, b2cat, rm2)
        sm = _combine_partials(_sc_scatter(payload, dst3, F), F)
        g = _sc_gather(sm, src4, F)
        newef = _pass2_gru(g, ef16, ief16, cw, out_3d=(step == 2))
        if step < 2:
            ef16 = newef

    return _readout(_sc_scatter(newef, dst3, F))[:N]
